# baseline (speedup 1.0000x reference)
import functools

import jax
import jax.numpy as jnp
from jax import lax
from jax.experimental import pallas as pl
from jax.experimental.pallas import tpu as pltpu
from jax.experimental.pallas import tpu_sc as plsc

_TIMESTEPS = 1000
_N = 16384
_NS = 16
_CHUNK = _N // _NS
_LANES = 16
_G = 1001

_mesh = plsc.VectorSubcoreMesh(
    core_axis_name="c", subcore_axis_name="s", num_cores=1, num_subcores=_NS
)


@functools.partial(
    pl.kernel,
    mesh=_mesh,
    out_type=jax.ShapeDtypeStruct((_N,), jnp.float32),
    compiler_params=pltpu.CompilerParams(needs_layout_passes=False),
    scratch_types=[
        pltpu.VMEM((_G,), jnp.float32),
        pltpu.VMEM((_CHUNK,), jnp.float32),
        pltpu.VMEM((_CHUNK,), jnp.float32),
        pltpu.SemaphoreType.DMA,
    ],
)
def _sc_probe(t_hbm, gamma_hbm, out_hbm, gamma_v, t_v, o_v, sem_t):
    base = lax.axis_index("s") * _CHUNK
    cp_t = pltpu.async_copy(t_hbm.at[pl.ds(base, _CHUNK)], t_v, sem_t)
    cp_t.wait()

    magic_f = jnp.float32(8388608.0)
    magic_i = jnp.int32(0x4B000000)

    def body(i, carry):
        x = t_v[pl.ds(i * _LANES, _LANES)] * jnp.float32(_TIMESTEPS)
        idx = plsc.bitcast(x + magic_f, jnp.int32) - magic_i
        o_v[pl.ds(i * _LANES, _LANES)] = plsc.load_gather(gamma_v, [idx])
        return carry

    lax.fori_loop(0, _CHUNK // _LANES, body, 0, unroll=8)
    pltpu.sync_copy(o_v, out_hbm.at[pl.ds(base, _CHUNK)])


def kernel(t, gamma):
    out = _sc_probe(t.reshape(_N), gamma)
    return out.reshape(t.shape)
